# baseline (speedup 1.0000x reference)
import functools

import jax
import jax.numpy as jnp
from jax import lax
from jax.experimental import pallas as pl
from jax.experimental.pallas import tpu as pltpu
from jax.experimental.pallas import tpu_sc as plsc

VOCAB = 1000000
SEQ = 200
BATCH = 16384
EMB = 3
TOTAL = BATCH * SEQ

NC = 2
NS = 16
NW = NC * NS
PER_W = TOTAL // NW
CHUNK = 6400
NCHUNK = PER_W // CHUNK
KSUB = 16
SUB = CHUNK // KSUB

BROWS = 800
NBCHUNK = VOCAB // BROWS

_SCP = pltpu.CompilerParams(
    use_tc_tiling_on_sc=False, needs_layout_passes=False
)


def _mesh():
  return plsc.VectorSubcoreMesh(
      core_axis_name="c", subcore_axis_name="s", num_cores=NC, num_subcores=NS
  )


_PAT = [(16 * m + l) // 3 for m in range(3) for l in range(16)] + [
    (16 * m + l) % 3 for m in range(3) for l in range(16)
]


@functools.cache
def _make_gather():
  out1d = jax.ShapeDtypeStruct((TOTAL,), jnp.float32)

  @functools.partial(
      pl.kernel,
      mesh=_mesh(),
      out_type=(out1d, out1d, out1d),
      scratch_types=[
          pltpu.HBM((VOCAB, 8), jnp.float32),
          pltpu.VMEM((96,), jnp.int32),
          pltpu.VMEM((3 * BROWS,), jnp.float32),
          pltpu.VMEM((BROWS, 8), jnp.float32),
          pltpu.VMEM((2, CHUNK), jnp.int32),
          pltpu.VMEM((CHUNK, 8), jnp.float32),
          pltpu.VMEM((CHUNK,), jnp.float32),
          pltpu.VMEM((CHUNK,), jnp.float32),
          pltpu.VMEM((CHUNK,), jnp.float32),
          pltpu.SemaphoreType.DMA,
          pltpu.SemaphoreType.DMA,
      ],
      compiler_params=_SCP,
  )
  def gather_kernel(pat_hbm, x_hbm, emb_flat_hbm,
                    g0_hbm, g1_hbm, g2_hbm,
                    emb_hbm, pat_v, buf3, rows8, idx_v, rows_v, v0, v1, v2,
                    sem, sem_idx):
    wid = lax.axis_index("s") * NC + lax.axis_index("c")
    sid = lax.axis_index("s")
    base = wid * PER_W
    iota = lax.iota(jnp.int32, 16)
    vals = (v0, v1, v2)

    pltpu.sync_copy(pat_hbm, pat_v)
    rp = [pat_v[pl.ds(16 * m, 16)] for m in range(3)]
    cp = [pat_v[pl.ds(48 + 16 * m, 16)] for m in range(3)]

    def bbody(j, _):
      c = j * NS + sid

      @pl.when(c < NBCHUNK)
      def _():
        pltpu.sync_copy(emb_flat_hbm.at[pl.ds(c * 3 * BROWS, 3 * BROWS)], buf3)

        def grp(g, _):
          for m in range(3):
            w16 = buf3[pl.ds(48 * g + 16 * m, 16)]
            plsc.store_scatter(rows8, [g * 16 + rp[m], cp[m]], w16)
          return 0

        lax.fori_loop(0, 3 * BROWS // 48, grp, 0)
        pltpu.sync_copy(rows8, emb_hbm.at[pl.ds(c * BROWS, BROWS), :])

      return 0

    lax.fori_loop(0, (NBCHUNK + NS - 1) // NS, bbody, 0)
    plsc.subcore_barrier()

    pltpu.sync_copy(x_hbm.at[pl.ds(base, CHUNK)], idx_v.at[0])

    def body(j, _):
      o = base + j * CHUNK
      sel = lax.rem(j, 2)
      cps = []
      for i in range(KSUB):
        cps.append(
            pltpu.async_copy(
                emb_hbm.at[idx_v.at[sel, pl.ds(i * SUB, SUB)]],
                rows_v.at[pl.ds(i * SUB, SUB), :],
                sem,
            )
        )
      @pl.when(j + 1 < NCHUNK)
      def _():
        pltpu.async_copy(
            x_hbm.at[pl.ds(o + CHUNK, CHUNK)],
            idx_v.at[lax.rem(j + 1, 2)],
            sem_idx,
        ).wait()

      for cp in cps:
        cp.wait()

      def grp(g, _):
        rows = g * 16 + iota
        for d in range(EMB):
          x16 = plsc.load_gather(rows_v, [rows, jnp.full((16,), d, jnp.int32)])
          vals[d][pl.ds(g * 16, 16)] = x16
        return 0

      lax.fori_loop(0, CHUNK // 16, grp, 0)
      for d in range(EMB):
        pltpu.sync_copy(vals[d], (g0_hbm, g1_hbm, g2_hbm)[d].at[pl.ds(o, CHUNK)])
      return 0

    lax.fori_loop(0, NCHUNK, body, 0)

  return gather_kernel


BB = 1024


def _mlp_body(g0_ref, g1_ref, g2_ref, w10_ref, w11_ref, w12_ref, b1_ref,
              w2_ref, b2_ref, w3_ref, b3_ref, o_ref):
  h = jnp.dot(g0_ref[...], w10_ref[...], preferred_element_type=jnp.float32)
  h += jnp.dot(g1_ref[...], w11_ref[...], preferred_element_type=jnp.float32)
  h += jnp.dot(g2_ref[...], w12_ref[...], preferred_element_type=jnp.float32)
  h = jnp.maximum(h + b1_ref[...], 0.0)
  h = jnp.dot(h, w2_ref[...], preferred_element_type=jnp.float32) + b2_ref[...]
  h = jnp.maximum(h, 0.0)
  z = jnp.dot(h, w3_ref[...], preferred_element_type=jnp.float32) + b3_ref[...]
  o_ref[...] = 1.0 / (1.0 + jnp.exp(-z))


def _mlp(g0, g1, g2, w10, w11, w12, b1, w2t, b2, w3t, b3):
  grid = BATCH // BB
  gspec = pl.BlockSpec((BB, SEQ), lambda i: (i, 0))
  full = lambda shape: pl.BlockSpec(shape, lambda i: (0, 0))
  return pl.pallas_call(
      _mlp_body,
      grid=(grid,),
      in_specs=[
          gspec, gspec, gspec,
          full((SEQ, 10)), full((SEQ, 10)), full((SEQ, 10)),
          full((1, 10)),
          full((10, 5)),
          full((1, 5)),
          full((5, 3)),
          full((1, 3)),
      ],
      out_specs=pl.BlockSpec((BB, 3), lambda i: (i, 0)),
      out_shape=jax.ShapeDtypeStruct((BATCH, 3), jnp.float32),
  )(g0, g1, g2, w10, w11, w12, b1, w2t, b2, w3t, b3)


@jax.jit
def kernel(x, emb, W1, b1, W2, b2, W3, b3):
  x_flat = x.astype(jnp.int32).reshape(TOTAL)
  g0, g1, g2 = _make_gather()(
      jnp.asarray(_PAT, jnp.int32), x_flat, emb.reshape(VOCAB * EMB)
  )
  w1r = W1.reshape(10, SEQ, EMB)
  return _mlp(
      g0.reshape(BATCH, SEQ),
      g1.reshape(BATCH, SEQ),
      g2.reshape(BATCH, SEQ),
      w1r[:, :, 0].T,
      w1r[:, :, 1].T,
      w1r[:, :, 2].T,
      b1.reshape(1, 10),
      W2.T,
      b2.reshape(1, 5),
      W3.T,
      b3.reshape(1, 3),
  )


# speedup vs baseline: 5.7416x; 5.7416x over previous
import functools

import jax
import jax.numpy as jnp
from jax import lax
from jax.experimental import pallas as pl
from jax.experimental.pallas import tpu as pltpu
from jax.experimental.pallas import tpu_sc as plsc

VOCAB = 1000000
SEQ = 200
BATCH = 16384
EMB = 3
TOTAL = BATCH * SEQ

NC = 2
NS = 16
NW = NC * NS
PER_W = TOTAL // NW
CHUNK = 6400
NCHUNK = PER_W // CHUNK
KSUB = 16
SUB = CHUNK // KSUB

BROWS = 800
NBCHUNK = VOCAB // BROWS

_SCP = pltpu.CompilerParams(
    use_tc_tiling_on_sc=False, needs_layout_passes=False
)


def _mesh():
  return plsc.VectorSubcoreMesh(
      core_axis_name="c", subcore_axis_name="s", num_cores=NC, num_subcores=NS
  )


_PAT = [(16 * m + l) // 3 for m in range(3) for l in range(16)] + [
    (16 * m + l) % 3 for m in range(3) for l in range(16)
]


@functools.cache
def _make_gather():
  out1d = jax.ShapeDtypeStruct((TOTAL,), jnp.float32)

  @functools.partial(
      pl.kernel,
      mesh=_mesh(),
      out_type=(out1d, out1d, out1d),
      scratch_types=[
          pltpu.HBM((VOCAB, 8), jnp.float32),
          pltpu.VMEM((BROWS,), jnp.float32),
          pltpu.VMEM((BROWS,), jnp.float32),
          pltpu.VMEM((BROWS,), jnp.float32),
          pltpu.VMEM((BROWS, 8), jnp.float32),
          pltpu.VMEM((2, CHUNK), jnp.int32),
          pltpu.VMEM((CHUNK, 8), jnp.float32),
          pltpu.VMEM((CHUNK,), jnp.float32),
          pltpu.VMEM((CHUNK,), jnp.float32),
          pltpu.VMEM((CHUNK,), jnp.float32),
          pltpu.SemaphoreType.DMA,
          pltpu.SemaphoreType.DMA,
      ],
      compiler_params=_SCP,
  )
  def gather_kernel(x_hbm, t0_hbm, t1_hbm, t2_hbm,
                    g0_hbm, g1_hbm, g2_hbm,
                    emb_hbm, tb0, tb1, tb2, rows8, idx_v, rows_v, v0, v1, v2,
                    sem, sem_idx):
    wid = lax.axis_index("s") * NC + lax.axis_index("c")
    sid = lax.axis_index("s")
    base = wid * PER_W
    iota = lax.iota(jnp.int32, 16)
    vals = (v0, v1, v2)
    tabs = (t0_hbm, t1_hbm, t2_hbm)

    def bbody(j, _):
      c = j * NS + sid

      @pl.when(c < NBCHUNK)
      def _():
        tbs = (tb0, tb1, tb2)
        for d in range(EMB):
          pltpu.sync_copy(tabs[d].at[pl.ds(c * BROWS, BROWS)], tbs[d])

        def grp(g, _):
          rows = g * 16 + iota
          for d in range(EMB):
            w16 = tbs[d][pl.ds(g * 16, 16)]
            plsc.store_scatter(rows8, [rows, jnp.full((16,), d, jnp.int32)], w16)
          return 0

        lax.fori_loop(0, BROWS // 16, grp, 0)
        pltpu.sync_copy(rows8, emb_hbm.at[pl.ds(c * BROWS, BROWS), :])

      return 0

    lax.fori_loop(0, (NBCHUNK + NS - 1) // NS, bbody, 0)
    plsc.subcore_barrier()

    pltpu.sync_copy(x_hbm.at[pl.ds(base, CHUNK)], idx_v.at[0])

    def body(j, _):
      o = base + j * CHUNK
      sel = lax.rem(j, 2)
      cps = []
      for i in range(KSUB):
        cps.append(
            pltpu.async_copy(
                emb_hbm.at[idx_v.at[sel, pl.ds(i * SUB, SUB)]],
                rows_v.at[pl.ds(i * SUB, SUB), :],
                sem,
            )
        )
      @pl.when(j + 1 < NCHUNK)
      def _():
        pltpu.async_copy(
            x_hbm.at[pl.ds(o + CHUNK, CHUNK)],
            idx_v.at[lax.rem(j + 1, 2)],
            sem_idx,
        ).wait()

      for cp in cps:
        cp.wait()

      def grp(g, _):
        rows = g * 16 + iota
        for d in range(EMB):
          x16 = plsc.load_gather(rows_v, [rows, jnp.full((16,), d, jnp.int32)])
          vals[d][pl.ds(g * 16, 16)] = x16
        return 0

      lax.fori_loop(0, CHUNK // 16, grp, 0)
      for d in range(EMB):
        pltpu.sync_copy(vals[d], (g0_hbm, g1_hbm, g2_hbm)[d].at[pl.ds(o, CHUNK)])
      return 0

    lax.fori_loop(0, NCHUNK, body, 0)

  return gather_kernel


BB = 1024


def _mlp_body(g0_ref, g1_ref, g2_ref, w10_ref, w11_ref, w12_ref, b1_ref,
              w2_ref, b2_ref, w3_ref, b3_ref, o_ref):
  h = jnp.dot(g0_ref[...], w10_ref[...], preferred_element_type=jnp.float32)
  h += jnp.dot(g1_ref[...], w11_ref[...], preferred_element_type=jnp.float32)
  h += jnp.dot(g2_ref[...], w12_ref[...], preferred_element_type=jnp.float32)
  h = jnp.maximum(h + b1_ref[...], 0.0)
  h = jnp.dot(h, w2_ref[...], preferred_element_type=jnp.float32) + b2_ref[...]
  h = jnp.maximum(h, 0.0)
  z = jnp.dot(h, w3_ref[...], preferred_element_type=jnp.float32) + b3_ref[...]
  o_ref[...] = 1.0 / (1.0 + jnp.exp(-z))


def _mlp(g0, g1, g2, w10, w11, w12, b1, w2t, b2, w3t, b3):
  grid = BATCH // BB
  gspec = pl.BlockSpec((BB, SEQ), lambda i: (i, 0))
  full = lambda shape: pl.BlockSpec(shape, lambda i: (0, 0))
  return pl.pallas_call(
      _mlp_body,
      grid=(grid,),
      in_specs=[
          gspec, gspec, gspec,
          full((SEQ, 10)), full((SEQ, 10)), full((SEQ, 10)),
          full((1, 10)),
          full((10, 5)),
          full((1, 5)),
          full((5, 3)),
          full((1, 3)),
      ],
      out_specs=pl.BlockSpec((BB, 3), lambda i: (i, 0)),
      out_shape=jax.ShapeDtypeStruct((BATCH, 3), jnp.float32),
  )(g0, g1, g2, w10, w11, w12, b1, w2t, b2, w3t, b3)


@jax.jit
def kernel(x, emb, W1, b1, W2, b2, W3, b3):
  x_flat = x.astype(jnp.int32).reshape(TOTAL)
  g0, g1, g2 = _make_gather()(
      x_flat, jnp.asarray(emb[:, 0]), jnp.asarray(emb[:, 1]),
      jnp.asarray(emb[:, 2])
  )
  w1r = W1.reshape(10, SEQ, EMB)
  return _mlp(
      g0.reshape(BATCH, SEQ),
      g1.reshape(BATCH, SEQ),
      g2.reshape(BATCH, SEQ),
      w1r[:, :, 0].T,
      w1r[:, :, 1].T,
      w1r[:, :, 2].T,
      b1.reshape(1, 10),
      W2.T,
      b2.reshape(1, 5),
      W3.T,
      b3.reshape(1, 3),
  )
